# Initial kernel scaffold; baseline (speedup 1.0000x reference)
import jax
import jax.numpy as jnp
from jax.experimental import pallas as pl


def kernel(out_embs, to_indices, query_rel, W):
    raise NotImplementedError("write your pallas kernel here")


# speedup vs baseline: 1.4696x; 1.4696x over previous
import functools

import jax
import jax.numpy as jnp
from jax import lax
from jax.experimental import pallas as pl
from jax.experimental.pallas import tpu as pltpu
from jax.experimental.pallas import tpu_sc as plsc

N_EMB = 8192
N_REL = 4096
D = 1024
L = 16
INV_SQRT_D = 1.0 / (D ** 0.5)


def _proj_body(q_ref, w_ref, e_ref, p_ref, v_scr):
    @pl.when(pl.program_id(0) == 0)
    def _():
        v_scr[...] = jnp.dot(w_ref[...], q_ref[...],
                             preferred_element_type=jnp.float32,
                             precision=lax.Precision.HIGHEST)

    p_ref[...] = jnp.dot(e_ref[...], v_scr[...],
                         preferred_element_type=jnp.float32,
                         precision=lax.Precision.HIGHEST)


_proj = pl.pallas_call(
    _proj_body,
    grid=(8,),
    in_specs=[
        pl.BlockSpec((D, 1), lambda i: (0, 0)),
        pl.BlockSpec((D, D), lambda i: (0, 0)),
        pl.BlockSpec((N_EMB // 8, D), lambda i: (i, 0)),
    ],
    out_specs=pl.BlockSpec((N_EMB // 8, 1), lambda i: (i, 0)),
    out_shape=jax.ShapeDtypeStruct((N_EMB, 1), jnp.float32),
    scratch_shapes=[pltpu.VMEM((D, 1), jnp.float32)],
)


_sc_mesh = plsc.VectorSubcoreMesh(
    core_axis_name="c", subcore_axis_name="s", num_cores=2, num_subcores=16)


@functools.partial(
    pl.kernel,
    out_type=jax.ShapeDtypeStruct((N_EMB,), jnp.float32),
    mesh=_sc_mesh,
    scratch_types=[
        pltpu.VMEM((N_EMB,), jnp.float32),
        pltpu.VMEM((N_EMB,), jnp.int32),
        pltpu.VMEM((N_REL,), jnp.float32),
        pltpu.VMEM((N_REL,), jnp.float32),
        pltpu.VMEM((N_REL,), jnp.float32),
        pltpu.VMEM((N_EMB,), jnp.float32),
    ],
    compiler_params=pltpu.CompilerParams(needs_layout_passes=False),
)
def _sc_middle(p_hbm, idx_hbm, coeff_hbm, p_v, idx_v, s_v, c_v, t_v, o_v):
    cid = lax.axis_index("c")
    sid = lax.axis_index("s")

    @pl.when(jnp.logical_and(cid == 0, sid == 0))
    def _():
        pltpu.sync_copy(p_hbm, p_v)
        pltpu.sync_copy(idx_hbm, idx_v)

        zeros = jnp.zeros((L,), jnp.float32)

        def zero_body(i, carry):
            s_v[pl.ds(i * L, L)] = zeros
            c_v[pl.ds(i * L, L)] = zeros
            return carry

        lax.fori_loop(0, N_REL // L, zero_body, 0)

        ones = jnp.full((L,), 1.0, jnp.float32)

        def scat_body(i, carry):
            iv = idx_v[pl.ds(i * L, L)]
            pv = p_v[pl.ds(i * L, L)]
            plsc.addupdate_scatter(s_v, [iv], pv)
            plsc.addupdate_scatter(c_v, [iv], ones)
            return carry

        lax.fori_loop(0, N_EMB // L, scat_body, 0)

        def score_body(i, mx):
            sv = s_v[pl.ds(i * L, L)]
            cv = jnp.maximum(c_v[pl.ds(i * L, L)], 1.0)
            sc = sv / cv * INV_SQRT_D
            t_v[pl.ds(i * L, L)] = sc
            c_v[pl.ds(i * L, L)] = cv
            return jnp.maximum(mx, sc)

        mx = lax.fori_loop(0, N_REL // L, score_body,
                           jnp.full((L,), -1e30, jnp.float32))
        mx_s = jnp.full((L,), jnp.max(mx))

        def exp_body(i, sm):
            e = jnp.exp(t_v[pl.ds(i * L, L)] - mx_s)
            t_v[pl.ds(i * L, L)] = e
            return sm + e

        sm = lax.fori_loop(0, N_REL // L, exp_body, zeros)
        inv_sum = ones / jnp.full((L,), jnp.sum(sm))

        def w_body(i, carry):
            t_v[pl.ds(i * L, L)] = (t_v[pl.ds(i * L, L)] * inv_sum
                                    / c_v[pl.ds(i * L, L)])
            return carry

        lax.fori_loop(0, N_REL // L, w_body, 0)

        def gather_body(i, carry):
            iv = idx_v[pl.ds(i * L, L)]
            o_v[pl.ds(i * L, L)] = plsc.load_gather(t_v, [iv])
            return carry

        lax.fori_loop(0, N_EMB // L, gather_body, 0)
        pltpu.sync_copy(o_v, coeff_hbm)


def _out_body(c_ref, e_ref, w_ref, o_ref, u_scr):
    @pl.when(pl.program_id(0) == 0)
    def _():
        u_scr[...] = jnp.zeros_like(u_scr)

    u_scr[...] += jnp.dot(c_ref[...], e_ref[...],
                          preferred_element_type=jnp.float32,
                          precision=lax.Precision.HIGHEST)

    @pl.when(pl.program_id(0) == 7)
    def _():
        o_ref[...] = jnp.dot(u_scr[...], w_ref[...],
                             preferred_element_type=jnp.float32,
                             precision=lax.Precision.HIGHEST)


_out_k = pl.pallas_call(
    _out_body,
    grid=(8,),
    in_specs=[
        pl.BlockSpec((1, N_EMB // 8), lambda i: (0, i)),
        pl.BlockSpec((N_EMB // 8, D), lambda i: (i, 0)),
        pl.BlockSpec((D, D), lambda i: (0, 0)),
    ],
    out_specs=pl.BlockSpec((1, D), lambda i: (0, 0)),
    out_shape=jax.ShapeDtypeStruct((1, D), jnp.float32),
    scratch_shapes=[pltpu.VMEM((1, D), jnp.float32)],
)


@jax.jit
def kernel(out_embs, to_indices, query_rel, W):
    q2d = query_rel.reshape(D, 1)
    p = _proj(q2d, W, out_embs)
    idx = to_indices.astype(jnp.int32)
    coeff = _sc_middle(p.reshape(N_EMB), idx)
    out = _out_k(coeff.reshape(1, N_EMB), out_embs, W)
    return out.reshape(D)


# speedup vs baseline: 2.0602x; 1.4019x over previous
import functools

import jax
import jax.numpy as jnp
from jax import lax
from jax.experimental import pallas as pl
from jax.experimental.pallas import tpu as pltpu
from jax.experimental.pallas import tpu_sc as plsc

N_EMB = 8192
N_REL = 4096
D = 1024
L = 16
INV_SQRT_D = 1.0 / (D ** 0.5)


def _proj_body(q_ref, w_ref, e_ref, p_ref, v_scr):
    @pl.when(pl.program_id(0) == 0)
    def _():
        v_scr[...] = jnp.dot(w_ref[...], q_ref[...],
                             preferred_element_type=jnp.float32,
                             precision=lax.Precision.DEFAULT)

    p_ref[...] = jnp.dot(e_ref[...], v_scr[...],
                         preferred_element_type=jnp.float32,
                         precision=lax.Precision.DEFAULT)


_proj = pl.pallas_call(
    _proj_body,
    grid=(8,),
    in_specs=[
        pl.BlockSpec((D, 1), lambda i: (0, 0)),
        pl.BlockSpec((D, D), lambda i: (0, 0)),
        pl.BlockSpec((N_EMB // 8, D), lambda i: (i, 0)),
    ],
    out_specs=pl.BlockSpec((N_EMB // 8, 1), lambda i: (i, 0)),
    out_shape=jax.ShapeDtypeStruct((N_EMB, 1), jnp.float32),
    scratch_shapes=[pltpu.VMEM((D, 1), jnp.float32)],
)


_sc_mesh = plsc.VectorSubcoreMesh(
    core_axis_name="c", subcore_axis_name="s", num_cores=2, num_subcores=16)


@functools.partial(
    pl.kernel,
    out_type=jax.ShapeDtypeStruct((N_EMB,), jnp.float32),
    mesh=_sc_mesh,
    scratch_types=[
        pltpu.VMEM((N_EMB,), jnp.float32),
        pltpu.VMEM((N_EMB,), jnp.int32),
        pltpu.VMEM((N_REL,), jnp.float32),
        pltpu.VMEM((N_REL,), jnp.float32),
        pltpu.VMEM((N_REL,), jnp.float32),
        pltpu.VMEM((N_EMB,), jnp.float32),
    ],
    compiler_params=pltpu.CompilerParams(needs_layout_passes=False),
)
def _sc_middle(p_hbm, idx_hbm, coeff_hbm, p_v, idx_v, s_v, c_v, t_v, o_v):
    cid = lax.axis_index("c")
    sid = lax.axis_index("s")

    @pl.when(jnp.logical_and(cid == 0, sid == 0))
    def _():
        pltpu.sync_copy(p_hbm, p_v)
        pltpu.sync_copy(idx_hbm, idx_v)

        zeros = jnp.zeros((L,), jnp.float32)

        def zero_body(i, carry):
            s_v[pl.ds(i * L, L)] = zeros
            c_v[pl.ds(i * L, L)] = zeros
            return carry

        lax.fori_loop(0, N_REL // L, zero_body, 0)

        ones = jnp.full((L,), 1.0, jnp.float32)

        def scat_body(i, carry):
            iv = idx_v[pl.ds(i * L, L)]
            pv = p_v[pl.ds(i * L, L)]
            plsc.addupdate_scatter(s_v, [iv], pv)
            plsc.addupdate_scatter(c_v, [iv], ones)
            return carry

        lax.fori_loop(0, N_EMB // L, scat_body, 0)

        def score_body(i, mx):
            sv = s_v[pl.ds(i * L, L)]
            cv = jnp.maximum(c_v[pl.ds(i * L, L)], 1.0)
            sc = sv / cv * INV_SQRT_D
            t_v[pl.ds(i * L, L)] = sc
            c_v[pl.ds(i * L, L)] = cv
            return jnp.maximum(mx, sc)

        mx = lax.fori_loop(0, N_REL // L, score_body,
                           jnp.full((L,), -1e30, jnp.float32))
        mx_s = jnp.full((L,), jnp.max(mx))

        def exp_body(i, sm):
            e = jnp.exp(t_v[pl.ds(i * L, L)] - mx_s)
            t_v[pl.ds(i * L, L)] = e
            return sm + e

        sm = lax.fori_loop(0, N_REL // L, exp_body, zeros)
        inv_sum = ones / jnp.full((L,), jnp.sum(sm))

        def w_body(i, carry):
            t_v[pl.ds(i * L, L)] = (t_v[pl.ds(i * L, L)] * inv_sum
                                    / c_v[pl.ds(i * L, L)])
            return carry

        lax.fori_loop(0, N_REL // L, w_body, 0)

        def gather_body(i, carry):
            iv = idx_v[pl.ds(i * L, L)]
            o_v[pl.ds(i * L, L)] = plsc.load_gather(t_v, [iv])
            return carry

        lax.fori_loop(0, N_EMB // L, gather_body, 0)
        pltpu.sync_copy(o_v, coeff_hbm)


def _out_body(c_ref, e_ref, w_ref, o_ref, u_scr):
    @pl.when(pl.program_id(0) == 0)
    def _():
        u_scr[...] = jnp.zeros_like(u_scr)

    u_scr[...] += jnp.dot(c_ref[...], e_ref[...],
                          preferred_element_type=jnp.float32,
                          precision=lax.Precision.DEFAULT)

    @pl.when(pl.program_id(0) == 7)
    def _():
        o_ref[...] = jnp.dot(u_scr[...], w_ref[...],
                             preferred_element_type=jnp.float32,
                             precision=lax.Precision.DEFAULT)


_out_k = pl.pallas_call(
    _out_body,
    grid=(8,),
    in_specs=[
        pl.BlockSpec((1, N_EMB // 8), lambda i: (0, i)),
        pl.BlockSpec((N_EMB // 8, D), lambda i: (i, 0)),
        pl.BlockSpec((D, D), lambda i: (0, 0)),
    ],
    out_specs=pl.BlockSpec((1, D), lambda i: (0, 0)),
    out_shape=jax.ShapeDtypeStruct((1, D), jnp.float32),
    scratch_shapes=[pltpu.VMEM((1, D), jnp.float32)],
)


@jax.jit
def kernel(out_embs, to_indices, query_rel, W):
    q2d = query_rel.reshape(D, 1)
    p = _proj(q2d, W, out_embs)
    idx = to_indices.astype(jnp.int32)
    coeff = _sc_middle(p.reshape(N_EMB), idx)
    out = _out_k(coeff.reshape(1, N_EMB), out_embs, W)
    return out.reshape(D)
